# baseline (speedup 1.0000x reference)
import functools

import jax
import jax.numpy as jnp
from jax import lax
from jax.experimental import pallas as pl
from jax.experimental.pallas import tpu as pltpu
from jax.experimental.pallas import tpu_sc as plsc

NC = 2
NS = 16
NW = NC * NS
CHUNK = 80
ZROWS = 32


def _sc_aggregate(table, src_flat, dst3, agg_rows):
    n_rows, d = table.shape
    per_w = src_flat.shape[0] // NW
    cpw = per_w // CHUNK
    stripe = agg_rows // NS

    mesh = plsc.VectorSubcoreMesh(core_axis_name="c", subcore_axis_name="s")

    @functools.partial(
        pl.kernel,
        mesh=mesh,
        compiler_params=pltpu.CompilerParams(use_tc_tiling_on_sc=False),
        out_type=jax.ShapeDtypeStruct((NC, agg_rows, d), jnp.float32),
        scratch_types=[
            pltpu.VMEM((per_w,), jnp.int32),
            pltpu.VMEM((cpw, CHUNK), jnp.int32),
            pltpu.VMEM((CHUNK, d), jnp.float32),
            pltpu.VMEM((CHUNK, d), jnp.float32),
            pltpu.VMEM((ZROWS, d), jnp.float32),
            pltpu.VMEM_SHARED((agg_rows, d), jnp.float32),
            pltpu.SemaphoreType.DMA,
            pltpu.SemaphoreType.DMA,
            pltpu.SemaphoreType.DMA,
        ],
    )
    def k(tab_hbm, src_hbm, dst_hbm, out_hbm,
          src_v, dst_v, rows_a, rows_b, zbuf, agg_sh, sem_a, sem_b, zsem):
        c = lax.axis_index("c")
        s = lax.axis_index("s")
        wid = s * NC + c

        cp_src = pltpu.async_copy(src_hbm.at[pl.ds(wid * per_w, per_w)],
                                  src_v, sem_a)
        cp_dst = pltpu.async_copy(dst_hbm.at[wid], dst_v, sem_b)

        @pl.loop(0, ZROWS)
        def _(r):
            @pl.loop(0, d // 16)
            def _(cc):
                zbuf.at[r, pl.ds(cc * 16, 16)][...] = jnp.zeros((16,), jnp.float32)

        stripe0 = s * stripe

        nz = stripe // ZROWS
        zc = [pltpu.async_copy(zbuf, agg_sh.at[pl.ds(stripe0 + t * ZROWS, ZROWS)],
                               zsem)
              for t in range(2)]
        for t in range(2, nz):
            zc[t % 2].wait()
            zc[t % 2] = pltpu.async_copy(
                zbuf, agg_sh.at[pl.ds(stripe0 + t * ZROWS, ZROWS)], zsem)
        zc[0].wait()
        zc[1].wait()

        cp_src.wait()
        cp_dst.wait()
        plsc.subcore_barrier()

        pltpu.async_copy(tab_hbm.at[src_v.at[pl.ds(0, CHUNK)]], rows_a, sem_a)

        @pl.loop(0, cpw // 2)
        def _(jh):
            j = jh * 2
            pltpu.async_copy(
                tab_hbm.at[src_v.at[pl.ds((j + 1) * CHUNK, CHUNK)]],
                rows_b, sem_b)
            pltpu.make_async_copy(
                tab_hbm.at[src_v.at[pl.ds(j * CHUNK, CHUNK)]],
                rows_a, sem_a).wait()
            pltpu.sync_copy(rows_a, agg_sh.at[dst_v.at[j]], add=True)

            @pl.when(j + 2 < cpw)
            def _():
                pltpu.async_copy(
                    tab_hbm.at[src_v.at[pl.ds((j + 2) * CHUNK, CHUNK)]],
                    rows_a, sem_a)
            pltpu.make_async_copy(
                tab_hbm.at[src_v.at[pl.ds((j + 1) * CHUNK, CHUNK)]],
                rows_b, sem_b).wait()
            pltpu.sync_copy(rows_b, agg_sh.at[dst_v.at[j + 1]], add=True)

        if cpw % 2:
            pltpu.make_async_copy(
                tab_hbm.at[src_v.at[pl.ds((cpw - 1) * CHUNK, CHUNK)]],
                rows_a, sem_a).wait()
            pltpu.sync_copy(rows_a, agg_sh.at[dst_v.at[cpw - 1]], add=True)

        plsc.subcore_barrier()
        pltpu.sync_copy(agg_sh.at[pl.ds(stripe0, stripe)],
                        out_hbm.at[c, pl.ds(stripe0, stripe)])

    return k(table, src_flat, dst3)


def _tc_layer(partials, w, b2d, n, blk):
    _, _, d = partials.shape

    def body(p_ref, w_ref, b_ref, h_ref):
        s = p_ref[0] + p_ref[1]
        acc = jnp.dot(s, w_ref[...], preferred_element_type=jnp.float32,
                      precision=lax.Precision.HIGHEST)
        h_ref[...] = jnp.maximum(acc + b_ref[...], 0.0)

    return pl.pallas_call(
        body,
        grid=(n // blk,),
        in_specs=[
            pl.BlockSpec((NC, blk, d), lambda i: (0, i, 0)),
            pl.BlockSpec((d, d), lambda i: (0, 0)),
            pl.BlockSpec((1, d), lambda i: (0, 0)),
        ],
        out_specs=pl.BlockSpec((blk, d), lambda i: (i, 0)),
        out_shape=jax.ShapeDtypeStruct((n, d), jnp.float32),
    )(partials, w, b2d)


def _tc_final(partials, w2, b2d, batch3, wc, bc2d, n, blk, g):
    _, _, d = partials.shape
    nblk = n // blk

    def body(p_ref, w_ref, b_ref, bat_ref, wc_ref, bc_ref, out_ref,
             pooled_acc, counts_acc):
        i = pl.program_id(0)

        @pl.when(i == 0)
        def _():
            pooled_acc[...] = jnp.zeros((g, d), jnp.float32)
            counts_acc[...] = jnp.zeros((1, g), jnp.float32)

        s = p_ref[0] + p_ref[1]
        h = jnp.maximum(
            jnp.dot(s, w_ref[...], preferred_element_type=jnp.float32,
                    precision=lax.Precision.HIGHEST)
            + b_ref[...], 0.0)
        bblk = bat_ref[...].reshape(1, blk)
        gid = lax.broadcasted_iota(jnp.int32, (g, blk), 0)
        oht = (gid == bblk).astype(jnp.float32)
        pooled_acc[...] += jnp.dot(oht, h, preferred_element_type=jnp.float32,
                                   precision=lax.Precision.HIGHEST)
        counts_acc[...] += jnp.sum(oht, axis=1)[None, :]

        @pl.when(i == nblk - 1)
        def _():
            counts = jnp.maximum(counts_acc[...].reshape(g, 1), 1.0)
            pooled = pooled_acc[...] / counts
            out_ref[...] = jnp.dot(
                pooled, wc_ref[...], preferred_element_type=jnp.float32,
                precision=lax.Precision.HIGHEST) + bc_ref[...]

    return pl.pallas_call(
        body,
        grid=(nblk,),
        in_specs=[
            pl.BlockSpec((NC, blk, d), lambda i: (0, i, 0)),
            pl.BlockSpec((d, d), lambda i: (0, 0)),
            pl.BlockSpec((1, d), lambda i: (0, 0)),
            pl.BlockSpec((1, 1, blk), lambda i: (i, 0, 0)),
            pl.BlockSpec((d, 1), lambda i: (0, 0)),
            pl.BlockSpec((1, 1), lambda i: (0, 0)),
        ],
        out_specs=pl.BlockSpec((g, 1), lambda i: (0, 0)),
        out_shape=jax.ShapeDtypeStruct((g, 1), jnp.float32),
        scratch_shapes=[
            pltpu.VMEM((g, d), jnp.float32),
            pltpu.VMEM((1, g), jnp.float32),
        ],
    )(partials, w2, b2d, batch3, wc, bc2d)


def kernel(x, edge_index, batch, W1, b1, W2, b2, Wc, bc):
    n, d = x.shape
    e = edge_index.shape[1]
    g = 512
    blk = 1000

    e_pad = -(-e // (NW * CHUNK)) * CHUNK * NW
    stripe = -(-(n // NS) // ZROWS) * ZROWS
    agg_rows = stripe * NS

    src = edge_index[0]
    dst = edge_index[1]
    pad = e_pad - e
    if pad:
        src = jnp.concatenate([src, jnp.zeros((pad,), src.dtype)])
        dst = jnp.concatenate([dst, jnp.full((pad,), n, dst.dtype)])
    per_w = e_pad // NW
    dst3 = dst.reshape(NW, per_w // CHUNK, CHUNK)

    b1r = b1.reshape(1, d)
    b2r = b2.reshape(1, d)
    bcr = bc.reshape(1, 1)
    batch3 = batch.reshape(n // blk, 1, blk)

    p1 = _sc_aggregate(x, src, dst3, agg_rows)
    h1 = _tc_layer(p1, W1, b1r, n, blk)
    p2 = _sc_aggregate(h1, src, dst3, agg_rows)
    out = _tc_final(p2, W2, b2r, batch3, Wc, bcr, n, blk, g)
    return out


# speedup vs baseline: 1.0766x; 1.0766x over previous
import functools

import jax
import jax.numpy as jnp
from jax import lax
from jax.experimental import pallas as pl
from jax.experimental.pallas import tpu as pltpu
from jax.experimental.pallas import tpu_sc as plsc

NC = 2
NS = 16
NW = NC * NS
CHUNK = 80
ZROWS = 32


def _sc_aggregate(table, src_flat, dst3, agg_rows):
    n_rows, d = table.shape
    per_w = src_flat.shape[0] // NW
    cpw = per_w // CHUNK
    stripe = agg_rows // NS

    mesh = plsc.VectorSubcoreMesh(core_axis_name="c", subcore_axis_name="s")

    @functools.partial(
        pl.kernel,
        mesh=mesh,
        compiler_params=pltpu.CompilerParams(use_tc_tiling_on_sc=False),
        out_type=jax.ShapeDtypeStruct((NC, agg_rows, d), jnp.float32),
        scratch_types=[
            pltpu.VMEM((per_w,), jnp.int32),
            pltpu.VMEM((cpw, CHUNK), jnp.int32),
            pltpu.VMEM((CHUNK, d), jnp.float32),
            pltpu.VMEM((CHUNK, d), jnp.float32),
            pltpu.VMEM((ZROWS, d), jnp.float32),
            pltpu.VMEM_SHARED((agg_rows, d), jnp.float32),
            pltpu.SemaphoreType.DMA,
            pltpu.SemaphoreType.DMA,
            pltpu.SemaphoreType.DMA,
        ],
    )
    def k(tab_hbm, src_hbm, dst_hbm, out_hbm,
          src_v, dst_v, rows_a, rows_b, zbuf, agg_sh, sem_a, sem_b, zsem):
        c = lax.axis_index("c")
        s = lax.axis_index("s")
        wid = s * NC + c

        cp_src = pltpu.async_copy(src_hbm.at[pl.ds(wid * per_w, per_w)],
                                  src_v, sem_a)
        cp_dst = pltpu.async_copy(dst_hbm.at[wid], dst_v, sem_b)

        @pl.loop(0, ZROWS)
        def _(r):
            @pl.loop(0, d // 16)
            def _(cc):
                zbuf.at[r, pl.ds(cc * 16, 16)][...] = jnp.zeros((16,), jnp.float32)

        stripe0 = s * stripe

        nz = stripe // ZROWS
        zc = [pltpu.async_copy(zbuf, agg_sh.at[pl.ds(stripe0 + t * ZROWS, ZROWS)],
                               zsem)
              for t in range(2)]
        for t in range(2, nz):
            zc[t % 2].wait()
            zc[t % 2] = pltpu.async_copy(
                zbuf, agg_sh.at[pl.ds(stripe0 + t * ZROWS, ZROWS)], zsem)
        zc[0].wait()
        zc[1].wait()

        cp_src.wait()
        cp_dst.wait()
        plsc.subcore_barrier()

        pltpu.async_copy(tab_hbm.at[src_v.at[pl.ds(0, CHUNK)]], rows_a, sem_a)

        @pl.loop(0, cpw // 2)
        def _(jh):
            j = jh * 2
            pltpu.async_copy(
                tab_hbm.at[src_v.at[pl.ds((j + 1) * CHUNK, CHUNK)]],
                rows_b, sem_b)
            pltpu.make_async_copy(
                tab_hbm.at[src_v.at[pl.ds(j * CHUNK, CHUNK)]],
                rows_a, sem_a).wait()
            pltpu.sync_copy(rows_a, agg_sh.at[dst_v.at[j]], add=True)

            @pl.when(j + 2 < cpw)
            def _():
                pltpu.async_copy(
                    tab_hbm.at[src_v.at[pl.ds((j + 2) * CHUNK, CHUNK)]],
                    rows_a, sem_a)
            pltpu.make_async_copy(
                tab_hbm.at[src_v.at[pl.ds((j + 1) * CHUNK, CHUNK)]],
                rows_b, sem_b).wait()
            pltpu.sync_copy(rows_b, agg_sh.at[dst_v.at[j + 1]], add=True)

        if cpw % 2:
            pltpu.make_async_copy(
                tab_hbm.at[src_v.at[pl.ds((cpw - 1) * CHUNK, CHUNK)]],
                rows_a, sem_a).wait()
            pltpu.sync_copy(rows_a, agg_sh.at[dst_v.at[cpw - 1]], add=True)

        plsc.subcore_barrier()
        pltpu.sync_copy(agg_sh.at[pl.ds(stripe0, stripe)],
                        out_hbm.at[c, pl.ds(stripe0, stripe)])

    return k(table, src_flat, dst3)


def _tc_layer(partials, w, b2d, n, blk):
    _, _, d = partials.shape

    def body(p_ref, w_ref, b_ref, h_ref):
        s = p_ref[0] + p_ref[1]
        acc = jnp.dot(s, w_ref[...], preferred_element_type=jnp.float32)
        h_ref[...] = jnp.maximum(acc + b_ref[...], 0.0)

    return pl.pallas_call(
        body,
        grid=(n // blk,),
        in_specs=[
            pl.BlockSpec((NC, blk, d), lambda i: (0, i, 0)),
            pl.BlockSpec((d, d), lambda i: (0, 0)),
            pl.BlockSpec((1, d), lambda i: (0, 0)),
        ],
        out_specs=pl.BlockSpec((blk, d), lambda i: (i, 0)),
        out_shape=jax.ShapeDtypeStruct((n, d), jnp.float32),
    )(partials, w, b2d)


def _tc_final(partials, w2, b2d, batch3, wc, bc2d, n, blk, g):
    _, _, d = partials.shape
    nblk = n // blk

    def body(p_ref, w_ref, b_ref, bat_ref, wc_ref, bc_ref, out_ref,
             pooled_acc, counts_acc):
        i = pl.program_id(0)

        @pl.when(i == 0)
        def _():
            pooled_acc[...] = jnp.zeros((g, d), jnp.float32)
            counts_acc[...] = jnp.zeros((1, g), jnp.float32)

        s = p_ref[0] + p_ref[1]
        h = jnp.maximum(
            jnp.dot(s, w_ref[...], preferred_element_type=jnp.float32)
            + b_ref[...], 0.0)
        bblk = bat_ref[...].reshape(1, blk)
        gid = lax.broadcasted_iota(jnp.int32, (g, blk), 0)
        oht = (gid == bblk).astype(jnp.float32)
        pooled_acc[...] += jnp.dot(oht, h, preferred_element_type=jnp.float32,
                                   precision=lax.Precision.HIGHEST)
        counts_acc[...] += jnp.sum(oht, axis=1)[None, :]

        @pl.when(i == nblk - 1)
        def _():
            counts = jnp.maximum(counts_acc[...].reshape(g, 1), 1.0)
            pooled = pooled_acc[...] / counts
            out_ref[...] = jnp.dot(
                pooled, wc_ref[...], preferred_element_type=jnp.float32
            ) + bc_ref[...]

    return pl.pallas_call(
        body,
        grid=(nblk,),
        in_specs=[
            pl.BlockSpec((NC, blk, d), lambda i: (0, i, 0)),
            pl.BlockSpec((d, d), lambda i: (0, 0)),
            pl.BlockSpec((1, d), lambda i: (0, 0)),
            pl.BlockSpec((1, 1, blk), lambda i: (i, 0, 0)),
            pl.BlockSpec((d, 1), lambda i: (0, 0)),
            pl.BlockSpec((1, 1), lambda i: (0, 0)),
        ],
        out_specs=pl.BlockSpec((g, 1), lambda i: (0, 0)),
        out_shape=jax.ShapeDtypeStruct((g, 1), jnp.float32),
        scratch_shapes=[
            pltpu.VMEM((g, d), jnp.float32),
            pltpu.VMEM((1, g), jnp.float32),
        ],
    )(partials, w2, b2d, batch3, wc, bc2d)


def kernel(x, edge_index, batch, W1, b1, W2, b2, Wc, bc):
    n, d = x.shape
    e = edge_index.shape[1]
    g = 512
    blk = 1000

    e_pad = -(-e // (NW * CHUNK)) * CHUNK * NW
    stripe = -(-(n // NS) // ZROWS) * ZROWS
    agg_rows = stripe * NS

    src = edge_index[0]
    dst = edge_index[1]
    pad = e_pad - e
    if pad:
        src = jnp.concatenate([src, jnp.zeros((pad,), src.dtype)])
        dst = jnp.concatenate([dst, jnp.full((pad,), n, dst.dtype)])
    per_w = e_pad // NW
    dst3 = dst.reshape(NW, per_w // CHUNK, CHUNK)

    b1r = b1.reshape(1, d)
    b2r = b2.reshape(1, d)
    bcr = bc.reshape(1, 1)
    batch3 = batch.reshape(n // blk, 1, blk)

    p1 = _sc_aggregate(x, src, dst3, agg_rows)
    h1 = _tc_layer(p1, W1, b1r, n, blk)
    p2 = _sc_aggregate(h1, src, dst3, agg_rows)
    out = _tc_final(p2, W2, b2r, batch3, Wc, bcr, n, blk, g)
    return out
